# Initial kernel scaffold; baseline (speedup 1.0000x reference)
import jax
import jax.numpy as jnp
from jax.experimental import pallas as pl


def kernel(x, edge_index, W, b):
    raise NotImplementedError("write your pallas kernel here")


# speedup vs baseline: 14.9817x; 14.9817x over previous
import functools

import jax
import jax.numpy as jnp
from jax import lax
from jax.experimental import pallas as pl
from jax.experimental.pallas import tpu as pltpu
from jax.experimental.pallas import tpu_sc as plsc

_NC = 2
_NS = 16
_CW = 128
_TCB = 1000


def _edge_geom(N, E):
  EP = E // _NS
  RTF = EP // _CW
  TAIL = EP - RTF * _CW
  RTP = RTF + (1 if TAIL else 0)
  NRS = (N // _NS) // 8 * 8
  NRL = N - NRS * (_NS - 1)
  return EP, RTF, TAIL, RTP, NRS, NRL


def _make_deg(N, E):
  _, RTF, TAIL, RTP, NRS, NRL = _edge_geom(N, E)
  ZR = 128
  mesh = plsc.VectorSubcoreMesh(core_axis_name="c", subcore_axis_name="s")

  @functools.partial(
      pl.kernel,
      out_type=jax.ShapeDtypeStruct((N, 16), jnp.float32),
      mesh=mesh,
      scratch_types=[
          pltpu.VMEM_SHARED((N, 16), jnp.float32),
          pltpu.VMEM((RTP, _CW), jnp.int32),
          pltpu.VMEM((_CW, 16), jnp.float32),
          pltpu.VMEM((ZR, 16), jnp.float32),
      ],
  )
  def deg_kernel(dst_hbm, tab_hbm, tab_s, idx_v, ones_v, zero_v):
    c = lax.axis_index("c")
    s = lax.axis_index("s")

    @pl.when(c == 0)
    def _():
      one = jnp.full((16,), 1.0, jnp.float32)
      zero = jnp.zeros((16,), jnp.float32)

      def fill(i, carry):
        ones_v[i] = one
        zero_v[i] = zero
        return carry
      lax.fori_loop(0, _CW, fill, 0)

      def zchunk(f, carry):
        pltpu.sync_copy(zero_v, tab_s.at[pl.ds(s * NRS + f * ZR, ZR)])
        return carry
      lax.fori_loop(0, NRL // ZR, zchunk, 0)

      plsc.subcore_barrier()
      pltpu.sync_copy(dst_hbm.at[s], idx_v)

      def chunk(j, carry):
        pltpu.sync_copy(ones_v, tab_s.at[idx_v.at[j]], add=True)
        return carry
      lax.fori_loop(0, RTF, chunk, 0)

      for t in range(TAIL // 16):
        dt = idx_v[RTF, pl.ds(t * 16, 16)]
        pltpu.sync_copy(ones_v.at[pl.ds(0, 16)], tab_s.at[dt], add=True)

      plsc.subcore_barrier()
      pltpu.sync_copy(tab_s.at[pl.ds(s * NRS, NRL)],
                      tab_hbm.at[pl.ds(s * NRS, NRL)])

  return deg_kernel


def _make_tc(N, D, DH):
  B = _TCB
  NB = N // B

  def tc_body(x_ref, w_ref, tab_ref, g_ref, dinv_ref):
    deg = tab_ref[:, 0] + 1.0
    dv = lax.rsqrt(deg)
    h = jnp.dot(x_ref[...], w_ref[...], preferred_element_type=jnp.float32)
    g_ref[...] = h * dv[:, None]
    dinv_ref[...] = jnp.broadcast_to(dv[:, None], dv.shape + (16,))

  return pl.pallas_call(
      tc_body,
      grid=(_NC, NB),
      in_specs=[
          pl.BlockSpec((B, D), lambda h, i: (i, 0)),
          pl.BlockSpec((D, DH), lambda h, i: (0, h)),
          pl.BlockSpec((B, 16), lambda h, i: (i, 0)),
      ],
      out_specs=[
          pl.BlockSpec((B, DH), lambda h, i: (h * NB + i, 0)),
          pl.BlockSpec((B, 16), lambda h, i: (i, 0)),
      ],
      out_shape=[
          jax.ShapeDtypeStruct((_NC * N, DH), jnp.float32),
          jax.ShapeDtypeStruct((N, 16), jnp.float32),
      ],
      compiler_params=pltpu.CompilerParams(
          dimension_semantics=("arbitrary", "arbitrary")),
  )


def _make_agg(N, E, DH):
  _, RTF, TAIL, RTP, NRS, NRL = _edge_geom(N, E)
  RC = 40
  NF = NRL // RC
  mesh = plsc.VectorSubcoreMesh(core_axis_name="c", subcore_axis_name="s")

  @functools.partial(
      pl.kernel,
      out_type=jax.ShapeDtypeStruct((_NC, N, DH), jnp.float32),
      mesh=mesh,
      scratch_types=[
          pltpu.VMEM_SHARED((N, DH), jnp.float32),
          pltpu.VMEM((RTP, _CW), jnp.int32),
          pltpu.VMEM((RTP, _CW), jnp.int32),
          pltpu.VMEM((_CW, DH), jnp.float32),
          pltpu.VMEM((RC, DH), jnp.float32),
          pltpu.VMEM((RC, 16), jnp.float32),
          pltpu.VMEM((DH,), jnp.float32),
          pltpu.SemaphoreType.DMA,
      ],
  )
  def agg_kernel(g_hbm, src_hbm, dst_hbm, dinv_hbm, b_hbm, out_hbm,
                 acc_s, src_v, dst_v, row_v, fin_v, dinv_v, b_v, gsem):
    c = lax.axis_index("c")
    s = lax.axis_index("s")
    r0 = s * NRS

    def initchunk(f, carry):
      fr = r0 + f * RC
      pltpu.sync_copy(g_hbm.at[pl.ds(c * N + fr, RC)], fin_v)
      pltpu.sync_copy(fin_v, acc_s.at[pl.ds(fr, RC)])
      return carry
    lax.fori_loop(0, NRL // RC, initchunk, 0)

    pltpu.sync_copy(src_hbm.at[s], src_v)
    pltpu.sync_copy(dst_hbm.at[s], dst_v)
    pltpu.sync_copy(b_hbm.at[c], b_v)

    bias = c * N
    NV = _CW // 16

    def bias_row(t, carry):
      i = t // NV
      k = t - i * NV
      sl = pl.ds(pl.multiple_of(k * 16, 16), 16)
      src_v[i, sl] = src_v[i, sl] + bias
      return carry
    lax.fori_loop(0, RTP * NV, bias_row, 0)

    plsc.subcore_barrier()

    def chunk(j, carry):
      pltpu.async_copy(g_hbm.at[src_v.at[j]], row_v, gsem).wait()
      pltpu.sync_copy(row_v, acc_s.at[dst_v.at[j]], add=True)
      return carry
    lax.fori_loop(0, RTF, chunk, 0)

    for t in range(TAIL // 16):
      st = src_v[RTF, pl.ds(t * 16, 16)]
      dt = dst_v[RTF, pl.ds(t * 16, 16)]
      tbuf = row_v.at[pl.ds(0, 16)]
      pltpu.async_copy(g_hbm.at[st], tbuf, gsem).wait()
      pltpu.sync_copy(tbuf, acc_s.at[dt], add=True)

    plsc.subcore_barrier()

    bvs = [b_v[pl.ds(k * 16, 16)] for k in range(DH // 16)]

    def finchunk(f, carry):
      fr = r0 + f * RC
      pltpu.sync_copy(acc_s.at[pl.ds(fr, RC)], fin_v)
      pltpu.sync_copy(dinv_hbm.at[pl.ds(fr, RC)], dinv_v)

      def row(i, icarry):
        dv = dinv_v[i]
        for k in range(DH // 16):
          sl = pl.ds(k * 16, 16)
          fin_v[i, sl] = jnp.maximum(fin_v[i, sl] * dv + bvs[k], 0.0)
        return icarry
      lax.fori_loop(0, RC, row, 0)

      pltpu.sync_copy(fin_v, out_hbm.at[c, pl.ds(fr, RC)])
      return carry
    lax.fori_loop(0, NF, finchunk, 0)

  return agg_kernel


def kernel(x, edge_index, W, b):
  N, D = x.shape
  DO = W.shape[1]
  E = edge_index.shape[1]
  DH = DO // _NC
  EP, RTF, TAIL, RTP, _, _ = _edge_geom(N, E)
  ei = edge_index.astype(jnp.int32).reshape(2, _NS, EP)
  pad = RTP * _CW - EP
  if pad:
    ei = jnp.pad(ei, ((0, 0), (0, 0), (0, pad)))
  src2 = ei[0].reshape(_NS, RTP, _CW)
  dst2 = ei[1].reshape(_NS, RTP, _CW)
  b2 = b.reshape(_NC, DH)

  tab = _make_deg(N, E)(dst2)
  g, dinv = _make_tc(N, D, DH)(x, W, tab)
  out3 = _make_agg(N, E, DH)(g, src2, dst2, dinv, b2)
  return jnp.concatenate([out3[0], out3[1]], axis=1)
